# baseline (speedup 1.0000x reference)
import functools

import jax
import jax.numpy as jnp
import numpy as np
from jax import lax
from jax.experimental import pallas as pl
from jax.experimental.pallas import tpu as pltpu
from jax.experimental.pallas import tpu_sc as plsc

N = 10000
E = 320000
D = 128
H = 8
C = 16

NSC = 2
NSUB = 16
NW = NSC * NSUB
K = 64
NCHUNK = E // K
CPW = 158

RSTRIDE = 624
ZR = 48
NP8 = N // 8
NP8P = 1280

RB = 2000
EB = 8000

_GDN = lax.GatherDimensionNumbers(
    offset_dims=(), collapsed_slice_dims=(0,), start_index_map=(0,))


def _vgather(vec, idx):
    return lax.gather(vec, idx.reshape(16, 1), _GDN, (1,),
                      mode=lax.GatherScatterMode.PROMISE_IN_BOUNDS)


def _lane_splat(vec, h):
    idx = lax.iota(jnp.int32, 16) * 0 + h
    return _vgather(vec, idx)


def _shift8(vec):
    idx = (lax.iota(jnp.int32, 16) & 7) + 8
    return _vgather(vec, idx)


def _fill_seq(idx_ref, base, rows):
    for j in range(rows // 16):
        idx_ref[pl.ds(16 * j, 16)] = base + 16 * j + lax.iota(jnp.int32, 16)


def _zero_fill(ref, rows, cols):
    z = jnp.zeros((16,), jnp.float32)

    def row(i, _):
        for j in range(cols // 16):
            ref[i, pl.ds(16 * j, 16)] = z
        return 0

    lax.fori_loop(0, rows, row, 0)


def _edge_pass_body(src_h, dst_h, ae_h, xh_h, att_h, um_h, ue_h,
                    src_v, dst_v, drow_v, src2_v, dst2_v, drow2_v,
                    ae_v, gx_v, gs_v, gd_v, exb_v,
                    idx_z, idx_t, um_sh, ue_sh,
                    sem1, sem2, sem3, sem4, sem5, sem6, sem7, sem8):
    c = lax.axis_index("c")
    s = lax.axis_index("s")
    w = c * NSUB + s

    _zero_fill(gx_v, ZR, D)
    for k in range(RSTRIDE // ZR):
        zoff = s * RSTRIDE + k * ZR
        _fill_seq(idx_z, zoff, ZR)
        pltpu.async_copy(gx_v.at[pl.ds(0, ZR), :], um_sh.at[idx_z], sem1).wait()

    for k in range(5):
        _fill_seq(idx_t, s * 80 + 16 * k, 16)
        pltpu.async_copy(gx_v.at[pl.ds(0, 16), :], ue_sh.at[idx_t], sem2).wait()

    @pl.when(s == NSUB - 1)
    def _():
        tail = NSUB * RSTRIDE
        _fill_seq(idx_t, tail, N - tail)
        pltpu.async_copy(gx_v.at[pl.ds(0, N - tail), :],
                         um_sh.at[idx_t], sem1).wait()

    plsc.subcore_barrier()

    ids = ((src_v, dst_v, drow_v), (src2_v, dst2_v, drow2_v))

    lane = lax.iota(jnp.int32, 16)
    sel = [(1 - jnp.minimum(jnp.abs(lane - j), 1)).astype(jnp.float32)
           for j in range(8)]

    def _drain(dummy_src, dst_ref, sem):
        pltpu.make_async_copy(dummy_src, dst_ref, sem).wait()

    dk_hbm = um_h.at[c, pl.ds(0, K), :]
    dk8_hbm = ae_h.at[pl.ds(0, K // 8), :]
    dki_hbm = src_h.at[pl.ds(0, K)]

    def load_ids(t, b, sa, sb):
        base = t * K
        ca = pltpu.async_copy(src_h.at[pl.ds(base, K)], ids[b][0], sa)
        cb = pltpu.async_copy(dst_h.at[pl.ds(base, K)], ids[b][1], sb)
        return ca, cb

    def fire_ae(t):
        return pltpu.async_copy(
            ae_h.at[pl.ds(t * (K // 8), K // 8), :], ae_v, sem3)

    def fire_att(b):
        c1 = pltpu.async_copy(att_h.at[ids[b][0]], gs_v, sem2)
        c2 = pltpu.async_copy(att_h.at[ids[b][1]], gd_v, sem4)
        return c1, c2

    def fire_gx(b):
        return pltpu.async_copy(xh_h.at[ids[b][0]], gx_v, sem1)

    t0 = w * CPW
    ca, cb = load_ids(t0, 0, sem5, sem6)
    cae = fire_ae(t0)
    ca.wait()
    cb.wait()
    c1, c2 = fire_att(0)
    cg = fire_gx(0)

    def pair(go, _):
        for b in (0, 1):
            g = go * 2 + b
            t = w * CPW + g
            nb = 1 - b
            vg = t < NCHUNK
            vg1 = jnp.logical_and(g + 1 < CPW, t + 1 < NCHUNK)

            @pl.when(vg)
            def _():
                _drain(dk_hbm, gs_v, sem2)
                _drain(dk_hbm, gd_v, sem4)
                _drain(dk8_hbm, ae_v, sem3)
                for q in range(K // 16):
                    ids[b][2][pl.ds(16 * q, 16)] = (
                        ids[b][1][pl.ds(16 * q, 16)] >> 3)

                def grpa(q, _):
                    d7g = ids[b][1][pl.ds(16 * q, 16)] & 7
                    for m in range(16):
                        e = q * 16 + m
                        sv = gs_v[e, pl.ds(0, 16)]
                        dv = _shift8(gd_v[e, pl.ds(0, 16)])
                        aev = ae_v[e >> 3, pl.ds(16 * (e % 8), 16)]
                        t16 = sv + dv + aev
                        ex = jnp.exp(jnp.maximum(t16, 0.2 * t16))
                        gs_v[e, pl.ds(0, 16)] = ex
                        d7s = _lane_splat(d7g, m)
                        for j in range(7, -1, -1):
                            eqm = _vgather(sel[j], d7s)
                            exb_v[e, pl.ds(16 * j, 16)] = ex * eqm
                    return 0

                lax.fori_loop(0, K // 16, grpa, 0)
                pltpu.async_copy(exb_v, ue_sh.at[ids[b][2]], sem8, add=True)

            @pl.when(vg1)
            def _():
                load_ids(t + 1, nb, sem5, sem6)
                fire_ae(t + 1)

            @pl.when(vg)
            def _():
                _drain(dk_hbm, gx_v, sem1)

                def grpb(q, _):
                    for m in range(16):
                        e = q * 16 + m
                        ex = gs_v[e, pl.ds(0, 16)]
                        for h in range(H):
                            mlt = _lane_splat(ex, h)
                            gx_v[e, pl.ds(16 * h, 16)] = (
                                gx_v[e, pl.ds(16 * h, 16)] * mlt)
                    return 0

                lax.fori_loop(0, K // 16, grpb, 0)
                pltpu.async_copy(gx_v, um_sh.at[ids[b][1]], sem7, add=True)

            @pl.when(vg1)
            def _():
                _drain(dki_hbm, ids[nb][0], sem5)
                _drain(dki_hbm, ids[nb][1], sem6)
                fire_att(nb)

            @pl.when(vg)
            def _():
                _drain(dk_hbm, gx_v, sem7)

            @pl.when(vg1)
            def _():
                fire_gx(nb)

            @pl.when(vg)
            def _():
                _drain(dk_hbm, exb_v, sem8)

        return 0

    lax.fori_loop(0, CPW // 2, pair, 0)
    plsc.subcore_barrier()

    for k in range(RSTRIDE // ZR):
        off = s * RSTRIDE + k * ZR
        _fill_seq(idx_z, off, ZR)
        pltpu.async_copy(um_sh.at[idx_z], gx_v.at[pl.ds(0, ZR), :], sem1).wait()
        pltpu.sync_copy(gx_v.at[pl.ds(0, ZR), :], um_h.at[c, pl.ds(off, ZR), :])

    @pl.when(s == NSUB - 1)
    def _():
        tail = NSUB * RSTRIDE
        _fill_seq(idx_t, tail, N - tail)
        pltpu.async_copy(um_sh.at[idx_t],
                         gx_v.at[pl.ds(0, N - tail), :], sem1).wait()
        pltpu.sync_copy(gx_v.at[pl.ds(0, N - tail), :],
                        um_h.at[c, pl.ds(tail, N - tail), :])

    _fill_seq(idx_z, s * 80, 48)
    pltpu.async_copy(ue_sh.at[idx_z], gx_v.at[pl.ds(0, 48), :], sem2).wait()
    pltpu.sync_copy(gx_v.at[pl.ds(0, 48), :], ue_h.at[c, pl.ds(s * 80, 48), :])
    for k in range(2):
        _fill_seq(idx_t, s * 80 + 48 + 16 * k, 16)
        pltpu.async_copy(ue_sh.at[idx_t], gx_v.at[pl.ds(0, 16), :], sem2).wait()
        pltpu.sync_copy(gx_v.at[pl.ds(0, 16), :],
                        ue_h.at[c, pl.ds(s * 80 + 48 + 16 * k, 16), :])


_edge_pass = functools.partial(
    pl.kernel,
    out_type=(jax.ShapeDtypeStruct((NSC, N, D), jnp.float32),
              jax.ShapeDtypeStruct((NSC, NP8P, D), jnp.float32)),
    mesh=plsc.VectorSubcoreMesh(core_axis_name="c", subcore_axis_name="s"),
    scratch_types=[
        pltpu.VMEM((K,), jnp.int32),
        pltpu.VMEM((K,), jnp.int32),
        pltpu.VMEM((K,), jnp.int32),
        pltpu.VMEM((K,), jnp.int32),
        pltpu.VMEM((K,), jnp.int32),
        pltpu.VMEM((K,), jnp.int32),
        pltpu.VMEM((K // 8, D), jnp.float32),
        pltpu.VMEM((K, D), jnp.float32),
        pltpu.VMEM((K, D), jnp.float32),
        pltpu.VMEM((K, D), jnp.float32),
        pltpu.VMEM((K, D), jnp.float32),
        pltpu.VMEM((ZR,), jnp.int32),
        pltpu.VMEM((16,), jnp.int32),
        pltpu.VMEM_SHARED((N, D), jnp.float32),
        pltpu.VMEM_SHARED((NP8P, D), jnp.float32),
        pltpu.SemaphoreType.DMA,
        pltpu.SemaphoreType.DMA,
        pltpu.SemaphoreType.DMA,
        pltpu.SemaphoreType.DMA,
        pltpu.SemaphoreType.DMA,
        pltpu.SemaphoreType.DMA,
        pltpu.SemaphoreType.DMA,
        pltpu.SemaphoreType.DMA,
    ],
)(_edge_pass_body)


def _prep_body(x_ref, nW_ref, nb_ref, W_ref, Wr_ref,
               h_ref, xh_ref, att_ref):
    h = jnp.dot(x_ref[...], nW_ref[...], preferred_element_type=jnp.float32)
    h = h + nb_ref[...]
    h_ref[...] = h
    xh_ref[...] = jnp.dot(h, W_ref[...], preferred_element_type=jnp.float32)
    att_ref[...] = jnp.dot(h, Wr_ref[...], preferred_element_type=jnp.float32)


def _ae_body(ea_ref, M0_ref, c0_ref, M1_ref, c1_ref, ae0_ref, ae1_ref):
    ea = ea_ref[...]
    ae0_ref[...] = jnp.dot(ea, M0_ref[...], preferred_element_type=jnp.float32) + c0_ref[...]
    ae1_ref[...] = jnp.dot(ea, M1_ref[...], preferred_element_type=jnp.float32) + c1_ref[...]


_EYE8 = jnp.eye(8, dtype=jnp.float32)


def _combine(um0, um1, ue0, ue1, hprev, exp16, bias, lng, lnb, W1, b1, W2, b2):
    den = ue0 + ue1
    expand = jnp.dot(den, exp16, preferred_element_type=jnp.float32)
    u = um0 + um1
    out = u / (expand + 1e-16) + bias
    t = out + hprev
    mu = jnp.mean(t, axis=-1, keepdims=True)
    var = jnp.mean((t - mu) ** 2, axis=-1, keepdims=True)
    h2 = (t - mu) / jnp.sqrt(var + 1e-5) * lng + lnb
    f = jnp.maximum(jnp.dot(h2, W1, preferred_element_type=jnp.float32) + b1, 0.0)
    return jnp.dot(f, W2, preferred_element_type=jnp.float32) + b2


def _mid_body(um0_ref, um1_ref, ue0_ref, ue1_ref, h_ref, exp16_ref,
              bias_ref, lng_ref, lnb_ref, W1_ref, b1_ref, W2_ref, b2_ref,
              W_ref, Wr_ref,
              hn_ref, xh_ref, att_ref):
    h = _combine(um0_ref[...], um1_ref[...], ue0_ref[...], ue1_ref[...],
                 h_ref[...], exp16_ref[...], bias_ref[...], lng_ref[...],
                 lnb_ref[...], W1_ref[...], b1_ref[...], W2_ref[...], b2_ref[...])
    hn_ref[...] = h
    xh_ref[...] = jnp.dot(h, W_ref[...], preferred_element_type=jnp.float32)
    att_ref[...] = jnp.dot(h, Wr_ref[...], preferred_element_type=jnp.float32)


def _final_body(um0_ref, um1_ref, ue0_ref, ue1_ref, h_ref, exp16_ref,
                bias_ref, lng_ref, lnb_ref, W1_ref, b1_ref, W2_ref, b2_ref,
                out_ref):
    i = pl.program_id(0)
    h = _combine(um0_ref[...], um1_ref[...], ue0_ref[...], ue1_ref[...],
                 h_ref[...], exp16_ref[...], bias_ref[...], lng_ref[...],
                 lnb_ref[...], W1_ref[...], b1_ref[...], W2_ref[...], b2_ref[...])

    @pl.when(i == 0)
    def _():
        out_ref[...] = jnp.zeros_like(out_ref)

    out_ref[...] += jnp.sum(h, axis=0, keepdims=True) * (1.0 / N)


def _row_spec(cols):
    return pl.BlockSpec((RB, cols), lambda i: (i, 0))


def _full_spec(r, cols):
    return pl.BlockSpec((r, cols), lambda i: (0, 0))


def _fold(W3, att):
    return jnp.einsum('dhc,hc->dh', W3.reshape(D, H, C), att)


def kernel(x, edge_index, edge_attr, params):
    src = edge_index[0].astype(jnp.int32)
    dst = edge_index[1].astype(jnp.int32)

    Wrep, Mae, cae, Wmat = [], [], [], []
    for p in params['layers']:
        Me = _fold(p['We'], p['att_edge'])
        Mae.append(jnp.concatenate(
            [params['edge_W'] @ Me, jnp.zeros((16, H), jnp.float32)], axis=1))
        cae.append(jnp.concatenate(
            [params['edge_b'] @ Me, jnp.zeros((H,), jnp.float32)]).reshape(1, 2 * H))
        ws = _fold(p['W'], p['att_src'])
        wd = _fold(p['W'], p['att_dst'])
        Wrep.append(jnp.tile(jnp.concatenate([ws, wd], axis=1), (1, 8)))
        Wmat.append(p['W'])

    exp16 = np.zeros((C, D), np.float32)
    for h in range(H):
        exp16[h, h * C:(h + 1) * C] = 1.0
    exp16 = jnp.asarray(exp16)

    eap = edge_attr.reshape(E // 8, D)
    BD = [jnp.kron(_EYE8, Mae[0]), jnp.kron(_EYE8, Mae[1])]
    cb = [jnp.tile(cae[0], (1, 8)), jnp.tile(cae[1], (1, 8))]
    E8B = EB // 8
    ae = pl.pallas_call(
        _ae_body,
        grid=(E // EB,),
        in_specs=[pl.BlockSpec((E8B, D), lambda i: (i, 0)),
                  _full_spec(D, D), _full_spec(1, D),
                  _full_spec(D, D), _full_spec(1, D)],
        out_specs=[pl.BlockSpec((E8B, D), lambda i: (i, 0)),
                   pl.BlockSpec((E8B, D), lambda i: (i, 0))],
        out_shape=[jax.ShapeDtypeStruct((E // 8, D), jnp.float32),
                   jax.ShapeDtypeStruct((E // 8, D), jnp.float32)],
    )(eap, BD[0], cb[0], BD[1], cb[1])

    nb = params['node_b'].reshape(1, D)
    h0, xh, att = pl.pallas_call(
        _prep_body,
        grid=(N // RB,),
        in_specs=[_row_spec(D), _full_spec(D, D), _full_spec(1, D),
                  _full_spec(D, D), _full_spec(D, D)],
        out_specs=[_row_spec(D), _row_spec(D), _row_spec(D)],
        out_shape=[jax.ShapeDtypeStruct((N, D), jnp.float32),
                   jax.ShapeDtypeStruct((N, D), jnp.float32),
                   jax.ShapeDtypeStruct((N, D), jnp.float32)],
    )(x, params['node_W'], nb, Wmat[0], Wrep[0])

    h = h0
    for li, p in enumerate(params['layers']):
        um, ue = _edge_pass(src, dst, ae[li], xh, att)
        ue = ue.reshape(NSC, NP8P * 8, C)[:, :N, :]
        wargs = (exp16, p['bias'].reshape(1, D), p['ln_g'].reshape(1, D),
                 p['ln_b'].reshape(1, D), p['ff_W1'], p['ff_b1'].reshape(1, D),
                 p['ff_W2'], p['ff_b2'].reshape(1, D))
        warg_specs = [_full_spec(C, D)] + [_full_spec(1, D), _full_spec(1, D),
                                           _full_spec(1, D), _full_spec(D, D),
                                           _full_spec(1, D), _full_spec(D, D),
                                           _full_spec(1, D)]
        data_specs = [_row_spec(D), _row_spec(D), _row_spec(C), _row_spec(C),
                      _row_spec(D)]
        if li + 1 < len(params['layers']):
            h, xh, att = pl.pallas_call(
                _mid_body,
                grid=(N // RB,),
                in_specs=data_specs + warg_specs + [
                    _full_spec(D, D), _full_spec(D, D)],
                out_specs=[_row_spec(D), _row_spec(D), _row_spec(D)],
                out_shape=[jax.ShapeDtypeStruct((N, D), jnp.float32),
                           jax.ShapeDtypeStruct((N, D), jnp.float32),
                           jax.ShapeDtypeStruct((N, D), jnp.float32)],
            )(um[0], um[1], ue[0], ue[1], h, *wargs,
              Wmat[li + 1], Wrep[li + 1])
        else:
            out = pl.pallas_call(
                _final_body,
                grid=(N // RB,),
                in_specs=data_specs + warg_specs,
                out_specs=pl.BlockSpec((1, D), lambda i: (0, 0)),
                out_shape=jax.ShapeDtypeStruct((1, D), jnp.float32),
            )(um[0], um[1], ue[0], ue[1], h, *wargs)
    return out.reshape(D)


# speedup vs baseline: 1.1414x; 1.1414x over previous
import functools

import jax
import jax.numpy as jnp
import numpy as np
from jax import lax
from jax.experimental import pallas as pl
from jax.experimental.pallas import tpu as pltpu
from jax.experimental.pallas import tpu_sc as plsc

N = 10000
E = 320000
D = 128
H = 8
C = 16

NSC = 2
NSUB = 16
NW = NSC * NSUB
K = 64
NCHUNK = E // K
CPW = 158

RSTRIDE = 624
ZR = 48
NP8 = N // 8
NP8P = 1280

RB = 2000
EB = 8000

_GDN = lax.GatherDimensionNumbers(
    offset_dims=(), collapsed_slice_dims=(0,), start_index_map=(0,))


def _vgather(vec, idx):
    return lax.gather(vec, idx.reshape(16, 1), _GDN, (1,),
                      mode=lax.GatherScatterMode.PROMISE_IN_BOUNDS)


def _lane_splat(vec, h):
    idx = lax.iota(jnp.int32, 16) * 0 + h
    return _vgather(vec, idx)


def _shift8(vec):
    idx = (lax.iota(jnp.int32, 16) & 7) + 8
    return _vgather(vec, idx)


def _fill_seq(idx_ref, base, rows):
    for j in range(rows // 16):
        idx_ref[pl.ds(16 * j, 16)] = base + 16 * j + lax.iota(jnp.int32, 16)


def _zero_fill(ref, rows, cols):
    z = jnp.zeros((16,), jnp.float32)

    def row(i, _):
        for j in range(cols // 16):
            ref[i, pl.ds(16 * j, 16)] = z
        return 0

    lax.fori_loop(0, rows, row, 0)


def _edge_pass_body(src_h, dst_h, ae_h, xh_h, att_h, um_h, ue_h,
                    src_v, dst_v, drow_v, src2_v, dst2_v, drow2_v,
                    ae_v, gx_v, gs_v, gd_v, exb_v,
                    idx_z, idx_t, um_sh, ue_sh,
                    sem1, sem2, sem3, sem4, sem5, sem6, sem7, sem8):
    c = lax.axis_index("c")
    s = lax.axis_index("s")
    w = c * NSUB + s

    _zero_fill(gx_v, ZR, D)
    for k in range(RSTRIDE // ZR):
        zoff = s * RSTRIDE + k * ZR
        _fill_seq(idx_z, zoff, ZR)
        pltpu.async_copy(gx_v.at[pl.ds(0, ZR), :], um_sh.at[idx_z], sem1).wait()

    for k in range(5):
        _fill_seq(idx_t, s * 80 + 16 * k, 16)
        pltpu.async_copy(gx_v.at[pl.ds(0, 16), :], ue_sh.at[idx_t], sem2).wait()

    @pl.when(s == NSUB - 1)
    def _():
        tail = NSUB * RSTRIDE
        _fill_seq(idx_t, tail, N - tail)
        pltpu.async_copy(gx_v.at[pl.ds(0, N - tail), :],
                         um_sh.at[idx_t], sem1).wait()

    plsc.subcore_barrier()

    ids = ((src_v, dst_v, drow_v), (src2_v, dst2_v, drow2_v))

    lane = lax.iota(jnp.int32, 16)
    sel = [(1 - jnp.minimum(jnp.abs(lane - j), 1)).astype(jnp.float32)
           for j in range(8)]

    def _drain(dummy_src, dst_ref, sem):
        pltpu.make_async_copy(dummy_src, dst_ref, sem).wait()

    dk_hbm = um_h.at[c, pl.ds(0, K), :]
    dk8_hbm = ae_h.at[pl.ds(0, K // 8), :]
    dki_hbm = src_h.at[pl.ds(0, K)]

    def load_ids(t, b, sa, sb):
        base = t * K
        ca = pltpu.async_copy(src_h.at[pl.ds(base, K)], ids[b][0], sa)
        cb = pltpu.async_copy(dst_h.at[pl.ds(base, K)], ids[b][1], sb)
        return ca, cb

    def fire_ae(t):
        return pltpu.async_copy(
            ae_h.at[pl.ds(t * (K // 8), K // 8), :], ae_v, sem3)

    def fire_att_s(b):
        return pltpu.async_copy(att_h.at[ids[b][0]], gs_v, sem2)

    def fire_att_d(b):
        return pltpu.async_copy(att_h.at[ids[b][1]], gd_v, sem4)

    def fire_gx(b):
        return pltpu.async_copy(xh_h.at[ids[b][0]], gx_v, sem1)

    t0 = w * CPW
    ca, cb = load_ids(t0, 0, sem5, sem6)
    cae = fire_ae(t0)
    ca.wait()
    cb.wait()
    fire_att_s(0)
    fire_att_d(0)
    fire_gx(0)

    def pair(go, _):
        for b in (0, 1):
            g = go * 2 + b
            t = w * CPW + g
            nb = 1 - b
            vg = t < NCHUNK
            vg1 = jnp.logical_and(g + 1 < CPW, t + 1 < NCHUNK)

            @pl.when(vg1)
            def _():
                load_ids(t + 1, nb, sem5, sem6)

            @pl.when(vg)
            def _():
                _drain(dk_hbm, gd_v, sem4)
                _drain(dk_hbm, gs_v, sem2)
                _drain(dk8_hbm, ae_v, sem3)
                for q in range(K // 16):
                    ids[b][2][pl.ds(16 * q, 16)] = (
                        ids[b][1][pl.ds(16 * q, 16)] >> 3)

                def grpa(q, _):
                    d7g = ids[b][1][pl.ds(16 * q, 16)] & 7
                    for m in range(16):
                        e = q * 16 + m
                        sv = gs_v[e, pl.ds(0, 16)]
                        dv = _shift8(gd_v[e, pl.ds(0, 16)])
                        aev = ae_v[e >> 3, pl.ds(16 * (e % 8), 16)]
                        t16 = sv + dv + aev
                        ex = jnp.exp(jnp.maximum(t16, 0.2 * t16))
                        gs_v[e, pl.ds(0, 16)] = ex
                        d7s = _lane_splat(d7g, m)
                        for j in range(7, -1, -1):
                            eqm = _vgather(sel[j], d7s)
                            exb_v[e, pl.ds(16 * j, 16)] = ex * eqm
                    return 0

                lax.fori_loop(0, K // 16, grpa, 0)
                pltpu.async_copy(exb_v, ue_sh.at[ids[b][2]], sem8, add=True)

            @pl.when(vg1)
            def _():
                _drain(dki_hbm, ids[nb][0], sem5)
                _drain(dki_hbm, ids[nb][1], sem6)
                fire_ae(t + 1)
                fire_att_d(nb)

            @pl.when(vg)
            def _():
                _drain(dk_hbm, gx_v, sem1)

                def grpb(q, _):
                    for m in range(16):
                        e = q * 16 + m
                        ex = gs_v[e, pl.ds(0, 16)]
                        for h in range(H):
                            mlt = _lane_splat(ex, h)
                            gx_v[e, pl.ds(16 * h, 16)] = (
                                gx_v[e, pl.ds(16 * h, 16)] * mlt)
                    return 0

                lax.fori_loop(0, K // 16, grpb, 0)
                pltpu.async_copy(gx_v, um_sh.at[ids[b][1]], sem7, add=True)

            @pl.when(vg1)
            def _():
                fire_att_s(nb)

            @pl.when(vg)
            def _():
                _drain(dk_hbm, gx_v, sem7)

            @pl.when(vg1)
            def _():
                fire_gx(nb)

            @pl.when(vg)
            def _():
                _drain(dk_hbm, exb_v, sem8)

        return 0

    lax.fori_loop(0, CPW // 2, pair, 0)
    plsc.subcore_barrier()

    for k in range(RSTRIDE // ZR):
        off = s * RSTRIDE + k * ZR
        _fill_seq(idx_z, off, ZR)
        pltpu.async_copy(um_sh.at[idx_z], gx_v.at[pl.ds(0, ZR), :], sem1).wait()
        pltpu.sync_copy(gx_v.at[pl.ds(0, ZR), :], um_h.at[c, pl.ds(off, ZR), :])

    @pl.when(s == NSUB - 1)
    def _():
        tail = NSUB * RSTRIDE
        _fill_seq(idx_t, tail, N - tail)
        pltpu.async_copy(um_sh.at[idx_t],
                         gx_v.at[pl.ds(0, N - tail), :], sem1).wait()
        pltpu.sync_copy(gx_v.at[pl.ds(0, N - tail), :],
                        um_h.at[c, pl.ds(tail, N - tail), :])

    _fill_seq(idx_z, s * 80, 48)
    pltpu.async_copy(ue_sh.at[idx_z], gx_v.at[pl.ds(0, 48), :], sem2).wait()
    pltpu.sync_copy(gx_v.at[pl.ds(0, 48), :], ue_h.at[c, pl.ds(s * 80, 48), :])
    for k in range(2):
        _fill_seq(idx_t, s * 80 + 48 + 16 * k, 16)
        pltpu.async_copy(ue_sh.at[idx_t], gx_v.at[pl.ds(0, 16), :], sem2).wait()
        pltpu.sync_copy(gx_v.at[pl.ds(0, 16), :],
                        ue_h.at[c, pl.ds(s * 80 + 48 + 16 * k, 16), :])


_edge_pass = functools.partial(
    pl.kernel,
    out_type=(jax.ShapeDtypeStruct((NSC, N, D), jnp.float32),
              jax.ShapeDtypeStruct((NSC, NP8P, D), jnp.float32)),
    mesh=plsc.VectorSubcoreMesh(core_axis_name="c", subcore_axis_name="s"),
    scratch_types=[
        pltpu.VMEM((K,), jnp.int32),
        pltpu.VMEM((K,), jnp.int32),
        pltpu.VMEM((K,), jnp.int32),
        pltpu.VMEM((K,), jnp.int32),
        pltpu.VMEM((K,), jnp.int32),
        pltpu.VMEM((K,), jnp.int32),
        pltpu.VMEM((K // 8, D), jnp.float32),
        pltpu.VMEM((K, D), jnp.float32),
        pltpu.VMEM((K, D), jnp.float32),
        pltpu.VMEM((K, D), jnp.float32),
        pltpu.VMEM((K, D), jnp.float32),
        pltpu.VMEM((ZR,), jnp.int32),
        pltpu.VMEM((16,), jnp.int32),
        pltpu.VMEM_SHARED((N, D), jnp.float32),
        pltpu.VMEM_SHARED((NP8P, D), jnp.float32),
        pltpu.SemaphoreType.DMA,
        pltpu.SemaphoreType.DMA,
        pltpu.SemaphoreType.DMA,
        pltpu.SemaphoreType.DMA,
        pltpu.SemaphoreType.DMA,
        pltpu.SemaphoreType.DMA,
        pltpu.SemaphoreType.DMA,
        pltpu.SemaphoreType.DMA,
    ],
)(_edge_pass_body)


def _prep_body(x_ref, nW_ref, nb_ref, W_ref, Wr_ref,
               h_ref, xh_ref, att_ref):
    h = jnp.dot(x_ref[...], nW_ref[...], preferred_element_type=jnp.float32)
    h = h + nb_ref[...]
    h_ref[...] = h
    xh_ref[...] = jnp.dot(h, W_ref[...], preferred_element_type=jnp.float32)
    att_ref[...] = jnp.dot(h, Wr_ref[...], preferred_element_type=jnp.float32)


def _ae_body(ea_ref, M0_ref, c0_ref, M1_ref, c1_ref, ae0_ref, ae1_ref):
    ea = ea_ref[...]
    ae0_ref[...] = jnp.dot(ea, M0_ref[...], preferred_element_type=jnp.float32) + c0_ref[...]
    ae1_ref[...] = jnp.dot(ea, M1_ref[...], preferred_element_type=jnp.float32) + c1_ref[...]


_EYE8 = jnp.eye(8, dtype=jnp.float32)


def _combine(um0, um1, ue0, ue1, hprev, exp16, bias, lng, lnb, W1, b1, W2, b2):
    den = ue0 + ue1
    expand = jnp.dot(den, exp16, preferred_element_type=jnp.float32)
    u = um0 + um1
    out = u / (expand + 1e-16) + bias
    t = out + hprev
    mu = jnp.mean(t, axis=-1, keepdims=True)
    var = jnp.mean((t - mu) ** 2, axis=-1, keepdims=True)
    h2 = (t - mu) / jnp.sqrt(var + 1e-5) * lng + lnb
    f = jnp.maximum(jnp.dot(h2, W1, preferred_element_type=jnp.float32) + b1, 0.0)
    return jnp.dot(f, W2, preferred_element_type=jnp.float32) + b2


def _mid_body(um0_ref, um1_ref, ue0_ref, ue1_ref, h_ref, exp16_ref,
              bias_ref, lng_ref, lnb_ref, W1_ref, b1_ref, W2_ref, b2_ref,
              W_ref, Wr_ref,
              hn_ref, xh_ref, att_ref):
    h = _combine(um0_ref[...], um1_ref[...], ue0_ref[...], ue1_ref[...],
                 h_ref[...], exp16_ref[...], bias_ref[...], lng_ref[...],
                 lnb_ref[...], W1_ref[...], b1_ref[...], W2_ref[...], b2_ref[...])
    hn_ref[...] = h
    xh_ref[...] = jnp.dot(h, W_ref[...], preferred_element_type=jnp.float32)
    att_ref[...] = jnp.dot(h, Wr_ref[...], preferred_element_type=jnp.float32)


def _final_body(um0_ref, um1_ref, ue0_ref, ue1_ref, h_ref, exp16_ref,
                bias_ref, lng_ref, lnb_ref, W1_ref, b1_ref, W2_ref, b2_ref,
                out_ref):
    i = pl.program_id(0)
    h = _combine(um0_ref[...], um1_ref[...], ue0_ref[...], ue1_ref[...],
                 h_ref[...], exp16_ref[...], bias_ref[...], lng_ref[...],
                 lnb_ref[...], W1_ref[...], b1_ref[...], W2_ref[...], b2_ref[...])

    @pl.when(i == 0)
    def _():
        out_ref[...] = jnp.zeros_like(out_ref)

    out_ref[...] += jnp.sum(h, axis=0, keepdims=True) * (1.0 / N)


def _row_spec(cols):
    return pl.BlockSpec((RB, cols), lambda i: (i, 0))


def _full_spec(r, cols):
    return pl.BlockSpec((r, cols), lambda i: (0, 0))


def _fold(W3, att):
    return jnp.einsum('dhc,hc->dh', W3.reshape(D, H, C), att)


def kernel(x, edge_index, edge_attr, params):
    src = edge_index[0].astype(jnp.int32)
    dst = edge_index[1].astype(jnp.int32)

    Wrep, Mae, cae, Wmat = [], [], [], []
    for p in params['layers']:
        Me = _fold(p['We'], p['att_edge'])
        Mae.append(jnp.concatenate(
            [params['edge_W'] @ Me, jnp.zeros((16, H), jnp.float32)], axis=1))
        cae.append(jnp.concatenate(
            [params['edge_b'] @ Me, jnp.zeros((H,), jnp.float32)]).reshape(1, 2 * H))
        ws = _fold(p['W'], p['att_src'])
        wd = _fold(p['W'], p['att_dst'])
        Wrep.append(jnp.tile(jnp.concatenate([ws, wd], axis=1), (1, 8)))
        Wmat.append(p['W'])

    exp16 = np.zeros((C, D), np.float32)
    for h in range(H):
        exp16[h, h * C:(h + 1) * C] = 1.0
    exp16 = jnp.asarray(exp16)

    eap = edge_attr.reshape(E // 8, D)
    BD = [jnp.kron(_EYE8, Mae[0]), jnp.kron(_EYE8, Mae[1])]
    cb = [jnp.tile(cae[0], (1, 8)), jnp.tile(cae[1], (1, 8))]
    E8B = EB // 8
    ae = pl.pallas_call(
        _ae_body,
        grid=(E // EB,),
        in_specs=[pl.BlockSpec((E8B, D), lambda i: (i, 0)),
                  _full_spec(D, D), _full_spec(1, D),
                  _full_spec(D, D), _full_spec(1, D)],
        out_specs=[pl.BlockSpec((E8B, D), lambda i: (i, 0)),
                   pl.BlockSpec((E8B, D), lambda i: (i, 0))],
        out_shape=[jax.ShapeDtypeStruct((E // 8, D), jnp.float32),
                   jax.ShapeDtypeStruct((E // 8, D), jnp.float32)],
    )(eap, BD[0], cb[0], BD[1], cb[1])

    nb = params['node_b'].reshape(1, D)
    h0, xh, att = pl.pallas_call(
        _prep_body,
        grid=(N // RB,),
        in_specs=[_row_spec(D), _full_spec(D, D), _full_spec(1, D),
                  _full_spec(D, D), _full_spec(D, D)],
        out_specs=[_row_spec(D), _row_spec(D), _row_spec(D)],
        out_shape=[jax.ShapeDtypeStruct((N, D), jnp.float32),
                   jax.ShapeDtypeStruct((N, D), jnp.float32),
                   jax.ShapeDtypeStruct((N, D), jnp.float32)],
    )(x, params['node_W'], nb, Wmat[0], Wrep[0])

    h = h0
    for li, p in enumerate(params['layers']):
        um, ue = _edge_pass(src, dst, ae[li], xh, att)
        ue = ue.reshape(NSC, NP8P * 8, C)[:, :N, :]
        wargs = (exp16, p['bias'].reshape(1, D), p['ln_g'].reshape(1, D),
                 p['ln_b'].reshape(1, D), p['ff_W1'], p['ff_b1'].reshape(1, D),
                 p['ff_W2'], p['ff_b2'].reshape(1, D))
        warg_specs = [_full_spec(C, D)] + [_full_spec(1, D), _full_spec(1, D),
                                           _full_spec(1, D), _full_spec(D, D),
                                           _full_spec(1, D), _full_spec(D, D),
                                           _full_spec(1, D)]
        data_specs = [_row_spec(D), _row_spec(D), _row_spec(C), _row_spec(C),
                      _row_spec(D)]
        if li + 1 < len(params['layers']):
            h, xh, att = pl.pallas_call(
                _mid_body,
                grid=(N // RB,),
                in_specs=data_specs + warg_specs + [
                    _full_spec(D, D), _full_spec(D, D)],
                out_specs=[_row_spec(D), _row_spec(D), _row_spec(D)],
                out_shape=[jax.ShapeDtypeStruct((N, D), jnp.float32),
                           jax.ShapeDtypeStruct((N, D), jnp.float32),
                           jax.ShapeDtypeStruct((N, D), jnp.float32)],
            )(um[0], um[1], ue[0], ue[1], h, *wargs,
              Wmat[li + 1], Wrep[li + 1])
        else:
            out = pl.pallas_call(
                _final_body,
                grid=(N // RB,),
                in_specs=data_specs + warg_specs,
                out_specs=pl.BlockSpec((1, D), lambda i: (0, 0)),
                out_shape=jax.ShapeDtypeStruct((1, D), jnp.float32),
            )(um[0], um[1], ue[0], ue[1], h, *wargs)
    return out.reshape(D)
